# baseline (speedup 1.0000x reference)
import functools

import jax
import jax.numpy as jnp
from jax import lax
from jax.experimental import pallas as pl
from jax.experimental.pallas import tpu as pltpu
from jax.experimental.pallas import tpu_sc as plsc

_NC = 2
_NS = 16
_NW = _NC * _NS
_CH = 128


def _sc_gather(ids_flat, table):
    BS = ids_flat.shape[0]
    _, E = table.shape
    b_per_w = BS // _NW
    n_ch = b_per_w // _CH

    mesh = plsc.VectorSubcoreMesh(core_axis_name="c", subcore_axis_name="s")

    def body(ids_hbm, table_hbm, out_hbm, idx_v, rows_v, sem):
        wid = lax.axis_index("s") * _NC + lax.axis_index("c")
        base = wid * b_per_w
        pltpu.sync_copy(ids_hbm.at[wid], idx_v)
        copies = []
        for j in range(n_ch):
            copies.append(
                pltpu.async_copy(
                    table_hbm.at[idx_v.at[j]],
                    rows_v.at[pl.ds(j * _CH, _CH)],
                    sem,
                )
            )
        for cp in copies:
            cp.wait()
        pltpu.sync_copy(rows_v, out_hbm.at[pl.ds(base, b_per_w)])

    ids3 = ids_flat.reshape(_NW, n_ch, _CH)
    return pl.kernel(
        body,
        out_type=jax.ShapeDtypeStruct((BS, E), jnp.float32),
        mesh=mesh,
        scratch_types=[
            pltpu.VMEM((n_ch, _CH), jnp.int32),
            pltpu.VMEM((b_per_w, E), jnp.float32),
            pltpu.SemaphoreType.DMA,
        ],
    )(ids3, table)


def _tc_body_first(seg_ref, g_ref, pos_ref, segtab_ref, w_ref, b_ref, gm_ref,
                   bt_ref, o_ref):
    x = g_ref[...] + pos_ref[...]
    sid = seg_ref[...]
    x = x + jnp.where(sid == 1, segtab_ref[1:2, :], segtab_ref[0:1, :])
    y = jnp.dot(x, w_ref[...], preferred_element_type=jnp.float32) + b_ref[...]
    mu = jnp.mean(y, axis=-1, keepdims=True)
    var = jnp.mean((y - mu) ** 2, axis=-1, keepdims=True)
    o_ref[...] = (y - mu) * lax.rsqrt(var + 1e-5) * gm_ref[...] + bt_ref[...]


def _tc_body(seg_ref, g_ref, pos_ref, segtab_ref, w_ref, b_ref, gm_ref,
             bt_ref, acc_ref, o_ref):
    del acc_ref
    _tc_body_first(seg_ref, g_ref, pos_ref, segtab_ref, w_ref, b_ref, gm_ref,
                   bt_ref, o_ref)


def _tc_chunk(gathered_c, seg2_c, pos_s, seg_table, W, b2, gm2, bt2,
              BS, chunk_idx, T, acc):
    CHT, E = gathered_c.shape
    H = W.shape[1]
    NB = CHT // T
    base = chunk_idx * NB
    specs = [
        pl.BlockSpec((T, 1), lambda i: (i, 0)),
        pl.BlockSpec((T, E), lambda i: (i, 0)),
        pl.BlockSpec((T, E), lambda i: (i, 0)),
        pl.BlockSpec((2, E), lambda i: (0, 0)),
        pl.BlockSpec((E, H), lambda i: (0, 0)),
        pl.BlockSpec((1, H), lambda i: (0, 0)),
        pl.BlockSpec((1, H), lambda i: (0, 0)),
        pl.BlockSpec((1, H), lambda i: (0, 0)),
    ]
    args = [seg2_c, gathered_c, pos_s, seg_table, W, b2, gm2, bt2]
    out_spec = pl.BlockSpec((T, H), lambda i: (base + i, 0))
    if acc is None:
        return pl.pallas_call(
            _tc_body_first,
            out_shape=jax.ShapeDtypeStruct((BS, H), jnp.float32),
            grid=(NB,),
            in_specs=specs,
            out_specs=out_spec,
        )(*args)
    return pl.pallas_call(
        _tc_body,
        out_shape=jax.ShapeDtypeStruct((BS, H), jnp.float32),
        grid=(NB,),
        in_specs=specs + [pl.BlockSpec(memory_space=pl.ANY)],
        out_specs=out_spec,
        input_output_aliases={8: 0},
    )(*args, acc)


def kernel(token_ids, seg_ids, tok_table, pos_table, seg_table, W, b, gamma,
           beta):
    B, S = token_ids.shape
    H = W.shape[1]
    E = tok_table.shape[1]
    BS = B * S
    T = 2048
    ids_flat = token_ids.reshape(-1).astype(jnp.int32)
    seg2 = seg_ids.reshape(BS, 1).astype(jnp.int32)
    pos_s = pos_table[:S]
    b2 = b.reshape(1, H)
    gm2 = gamma.reshape(1, H)
    bt2 = beta.reshape(1, H)

    gathered = [
        _sc_gather(ids_flat[c * S:(c + 1) * S], tok_table) for c in range(B)
    ]
    out = None
    for c in range(B):
        out = _tc_chunk(gathered[c], seg2[c * S:(c + 1) * S], pos_s, seg_table,
                        W, b2, gm2, bt2, BS, c, T, out)
    return out.reshape(B, S, H)


# speedup vs baseline: 1.3379x; 1.3379x over previous
import jax
import jax.numpy as jnp
from jax import lax
from jax.experimental import pallas as pl
from jax.experimental.pallas import tpu as pltpu
from jax.experimental.pallas import tpu_sc as plsc

_NC = 2
_NS = 16
_NW = _NC * _NS
_CH = 128


def _sc_gather(ids_flat, table):
    BS = ids_flat.shape[0]
    _, E = table.shape
    b_per_w = BS // _NW
    n_ch = b_per_w // _CH

    mesh = plsc.VectorSubcoreMesh(core_axis_name="c", subcore_axis_name="s")

    def body(ids_hbm, table_hbm, out_hbm, idx_v, rows_v, gsem, wsem):
        wid = lax.axis_index("s") * _NC + lax.axis_index("c")
        base = wid * b_per_w
        pltpu.sync_copy(ids_hbm.at[wid], idx_v)
        gathers = []
        for j in range(n_ch):
            gathers.append(
                pltpu.async_copy(
                    table_hbm.at[idx_v.at[j]],
                    rows_v.at[pl.ds(j * _CH, _CH)],
                    gsem,
                )
            )
        writes = []
        for j in range(n_ch):
            gathers[j].wait()
            writes.append(
                pltpu.async_copy(
                    rows_v.at[pl.ds(j * _CH, _CH)],
                    out_hbm.at[pl.ds(base + j * _CH, _CH)],
                    wsem,
                )
            )
        for cp in writes:
            cp.wait()

    ids3 = ids_flat.reshape(_NW, n_ch, _CH)
    return pl.kernel(
        body,
        out_type=jax.ShapeDtypeStruct((BS, E), jnp.float32),
        mesh=mesh,
        scratch_types=[
            pltpu.VMEM((n_ch, _CH), jnp.int32),
            pltpu.VMEM((b_per_w, E), jnp.float32),
            pltpu.SemaphoreType.DMA,
            pltpu.SemaphoreType.DMA,
        ],
    )(ids3, table)


def _tc_body(seg_ref, g_ref, pos_ref, segtab_ref, w_ref, b_ref, gm_ref, bt_ref,
             o_ref):
    x = g_ref[...] + pos_ref[...]
    sid = seg_ref[...]
    x = x + jnp.where(sid == 1, segtab_ref[1:2, :], segtab_ref[0:1, :])
    y = jnp.dot(x, w_ref[...], preferred_element_type=jnp.float32) + b_ref[...]
    mu = jnp.mean(y, axis=-1, keepdims=True)
    var = jnp.mean((y - mu) ** 2, axis=-1, keepdims=True)
    o_ref[...] = (y - mu) * lax.rsqrt(var + 1e-5) * gm_ref[...] + bt_ref[...]


def _tc_fuse(gathered, seg_flat, pos_table, seg_table, W, b, gamma, beta, S, T):
    BS, E = gathered.shape
    H = W.shape[1]
    NB = BS // T
    SB = S // T
    seg2 = seg_flat.reshape(BS, 1)
    return pl.pallas_call(
        _tc_body,
        out_shape=jax.ShapeDtypeStruct((BS, H), jnp.float32),
        grid=(NB,),
        in_specs=[
            pl.BlockSpec((T, 1), lambda i: (i, 0)),
            pl.BlockSpec((T, E), lambda i: (i, 0)),
            pl.BlockSpec((T, E), lambda i: (i % SB, 0)),
            pl.BlockSpec((2, E), lambda i: (0, 0)),
            pl.BlockSpec((E, H), lambda i: (0, 0)),
            pl.BlockSpec((1, H), lambda i: (0, 0)),
            pl.BlockSpec((1, H), lambda i: (0, 0)),
            pl.BlockSpec((1, H), lambda i: (0, 0)),
        ],
        out_specs=pl.BlockSpec((T, H), lambda i: (i, 0)),
    )(seg2, gathered, pos_table[:S], seg_table, W, b.reshape(1, H),
      gamma.reshape(1, H), beta.reshape(1, H))


def kernel(token_ids, seg_ids, tok_table, pos_table, seg_table, W, b, gamma,
           beta):
    B, S = token_ids.shape
    H = W.shape[1]
    ids_flat = token_ids.reshape(-1).astype(jnp.int32)
    seg_flat = seg_ids.reshape(-1).astype(jnp.int32)
    gathered = _sc_gather(ids_flat, tok_table)
    out = _tc_fuse(gathered, seg_flat, pos_table, seg_table, W, b, gamma, beta,
                   S, 4096)
    return out.reshape(B, S, H)
